# Initial kernel scaffold; baseline (speedup 1.0000x reference)
import jax
import jax.numpy as jnp
from jax import lax
from jax.experimental import pallas as pl
from jax.experimental.pallas import tpu as pltpu
from jax.experimental.pallas import tpu_sc as plsc

B = 16384
NF = 26
V = 100000
D = 32
C = 16

NC = 2
NS = 16
L = 16
NW = NC * NS
NB = B // NW
NL = NB * NF
G = 128
NG = NL // G
SUPER = 8
NSUP = NG // SUPER
ROWS_BUF = SUPER * G


def _tab_kernel(xcat_hbm, xcont_hbm, table_hbm, gamma_hbm, beta_hbm,
                emb_hbm, bn_hbm, parts_hbm,
                xv, offv, gidx, rows, xc, pbuf, pacc, gv, bv, sem):
    c = lax.axis_index("c")
    s = lax.axis_index("s")
    wid = s * NC + c
    base = wid * NL

    pltpu.sync_copy(xcat_hbm.at[pl.ds(base, NL)], xv)

    iota = lax.iota(jnp.int32, L)
    for k in range(13):
        offv[pl.ds(k * L, L)] = lax.rem(iota + (k * L), NF) * V

    def idx_body(g, _):
        for k8 in range(SUPER):
            i = g * SUPER + k8
            x = xv[pl.ds(i * L, L)]
            off = offv[pl.ds(lax.rem(i, 13) * L, L)]
            gidx[g, pl.ds(k8 * L, L)] = x + off
        return 0

    lax.fori_loop(0, NG, idx_body, 0)

    pltpu.sync_copy(xcont_hbm.at[pl.ds(s * (NB * NC), NB * NC)], xc)

    def acc_body(i, carry):
        acc, acc2 = carry
        v = xc[i, :]
        return acc + v, acc2 + v * v

    zero = jnp.zeros((L,), jnp.float32)
    acc, acc2 = lax.fori_loop(0, NB * NC, acc_body, (zero, zero))
    pbuf[0, :] = acc
    pbuf[1, :] = acc2
    pltpu.sync_copy(pbuf, parts_hbm.at[c, s])
    plsc.subcore_barrier()

    pltpu.sync_copy(parts_hbm.at[c], pacc)
    tot = jnp.zeros((L,), jnp.float32)
    tot2 = jnp.zeros((L,), jnp.float32)
    for k in range(NS):
        tot = tot + pacc[k, 0, :]
        tot2 = tot2 + pacc[k, 1, :]
    inv_b = jnp.float32(1.0 / B)
    mean = tot * inv_b
    var = tot2 * inv_b - mean * mean
    x = var + jnp.float32(1e-5)
    i32 = lax.bitcast_convert_type(x, jnp.int32)
    i32 = jnp.int32(0x5F3759DF) - lax.shift_right_logical(i32, 1)
    y = lax.bitcast_convert_type(i32, jnp.float32)
    for _ in range(4):
        y = y * (jnp.float32(1.5) - jnp.float32(0.5) * x * y * y)
    pltpu.sync_copy(gamma_hbm, gv)
    pltpu.sync_copy(beta_hbm, bv)
    scale = gv[...] * y
    shift = bv[...] - mean * scale

    def bn_body(i, _):
        r = c * NB + i
        xc[r, :] = xc[r, :] * scale + shift
        return 0

    lax.fori_loop(0, NB, bn_body, 0)
    pltpu.sync_copy(xc.at[pl.ds(c * NB, NB)], bn_hbm.at[pl.ds(wid * NB, NB)])

    def gather_body(sc_i, _):
        hs = []
        for g in range(SUPER):
            hs.append(pltpu.async_copy(
                table_hbm.at[gidx.at[sc_i * SUPER + g]],
                rows.at[pl.ds(g * G, G)], sem))
        for h in hs:
            h.wait()
        pltpu.sync_copy(rows, emb_hbm.at[pl.ds(base + sc_i * ROWS_BUF, ROWS_BUF)])
        return 0

    lax.fori_loop(0, NSUP, gather_body, 0)


@jax.jit
def _run(xcat_flat, x_cont, table, gamma, beta):
    mesh = plsc.VectorSubcoreMesh(core_axis_name="c", subcore_axis_name="s")
    f = pl.kernel(
        _tab_kernel,
        mesh=mesh,
        out_type=[
            jax.ShapeDtypeStruct((B * NF, D), jnp.float32),
            jax.ShapeDtypeStruct((B, C), jnp.float32),
            jax.ShapeDtypeStruct((NC, NS, 2, L), jnp.float32),
        ],
        scratch_types=[
            pltpu.VMEM((NL,), jnp.int32),
            pltpu.VMEM((13 * L,), jnp.int32),
            pltpu.VMEM((NG, G), jnp.int32),
            pltpu.VMEM((ROWS_BUF, D), jnp.float32),
            pltpu.VMEM((NB * NC, C), jnp.float32),
            pltpu.VMEM((2, L), jnp.float32),
            pltpu.VMEM((NS, 2, L), jnp.float32),
            pltpu.VMEM((L,), jnp.float32),
            pltpu.VMEM((L,), jnp.float32),
            pltpu.SemaphoreType.DMA,
        ],
    )
    emb, bn, _ = f(xcat_flat, x_cont, table, gamma, beta)
    return jnp.concatenate([emb.reshape(B, NF * D), bn], axis=1)


def kernel(x_cat, x_cont, emb_tables, bn_gamma, bn_beta):
    xcat_flat = x_cat.astype(jnp.int32).reshape(B * NF)
    table = emb_tables.reshape(NF * V, D)
    return _run(xcat_flat, x_cont, table, bn_gamma, bn_beta)


# speedup vs baseline: 1.1623x; 1.1623x over previous
import jax
import jax.numpy as jnp
from jax import lax
from jax.experimental import pallas as pl
from jax.experimental.pallas import tpu as pltpu
from jax.experimental.pallas import tpu_sc as plsc

B = 16384
NF = 26
V = 100000
D = 32
C = 16

NC = 2
NS = 16
L = 16
NW = NC * NS
NB = B // NW
NL = NB * NF
G = 128
NG = NL // G
SUPER = 8
NSUP = NG // SUPER
ROWS_BUF = SUPER * G


def _tab_kernel(xcat_hbm, xcont_hbm, table_hbm, gamma_hbm, beta_hbm,
                emb_hbm, bn_hbm, parts_hbm,
                xv, offv, gidx, rows, xc, pbuf, pacc, gv, bv, sem):
    c = lax.axis_index("c")
    s = lax.axis_index("s")
    wid = s * NC + c
    base = wid * NL

    pltpu.sync_copy(xcat_hbm.at[pl.ds(base, NL)], xv)

    iota = lax.iota(jnp.int32, L)
    for k in range(13):
        offv[pl.ds(k * L, L)] = lax.rem(iota + (k * L), NF) * V

    def idx_body(g, _):
        for k8 in range(SUPER):
            i = g * SUPER + k8
            x = xv[pl.ds(i * L, L)]
            off = offv[pl.ds(lax.rem(i, 13) * L, L)]
            gidx[g, pl.ds(k8 * L, L)] = x + off
        return 0

    lax.fori_loop(0, NG, idx_body, 0)

    pltpu.sync_copy(xcont_hbm.at[pl.ds(s * (NB * NC), NB * NC)], xc)

    def acc_body(i, carry):
        acc, acc2 = carry
        v = xc[i, :]
        return acc + v, acc2 + v * v

    zero = jnp.zeros((L,), jnp.float32)
    acc, acc2 = lax.fori_loop(0, NB * NC, acc_body, (zero, zero))
    pbuf[0, :] = acc
    pbuf[1, :] = acc2
    pltpu.sync_copy(pbuf, parts_hbm.at[c, s])
    plsc.subcore_barrier()

    pltpu.sync_copy(parts_hbm.at[c], pacc)
    tot = jnp.zeros((L,), jnp.float32)
    tot2 = jnp.zeros((L,), jnp.float32)
    for k in range(NS):
        tot = tot + pacc[k, 0, :]
        tot2 = tot2 + pacc[k, 1, :]
    inv_b = jnp.float32(1.0 / B)
    mean = tot * inv_b
    var = tot2 * inv_b - mean * mean
    x = var + jnp.float32(1e-5)
    i32 = lax.bitcast_convert_type(x, jnp.int32)
    i32 = jnp.int32(0x5F3759DF) - lax.shift_right_logical(i32, 1)
    y = lax.bitcast_convert_type(i32, jnp.float32)
    for _ in range(4):
        y = y * (jnp.float32(1.5) - jnp.float32(0.5) * x * y * y)
    pltpu.sync_copy(gamma_hbm, gv)
    pltpu.sync_copy(beta_hbm, bv)
    scale = gv[...] * y
    shift = bv[...] - mean * scale

    def bn_body(i, _):
        r = c * NB + i
        xc[r, :] = xc[r, :] * scale + shift
        return 0

    lax.fori_loop(0, NB, bn_body, 0)
    pltpu.sync_copy(xc.at[pl.ds(c * NB, NB)], bn_hbm.at[pl.ds(wid * NB, NB)])

    def gather_body(sc_i, _):
        hs = []
        for g in range(SUPER):
            hs.append(pltpu.async_copy(
                table_hbm.at[gidx.at[sc_i * SUPER + g]],
                rows.at[pl.ds(g * G, G)], sem))
        for h in hs:
            h.wait()
        pltpu.sync_copy(rows, emb_hbm.at[pl.ds(base + sc_i * ROWS_BUF, ROWS_BUF)])
        return 0

    lax.fori_loop(0, NSUP, gather_body, 0)


@jax.jit
def _run(xcat_flat, x_cont, table, gamma, beta):
    mesh = plsc.VectorSubcoreMesh(core_axis_name="c", subcore_axis_name="s")
    f = pl.kernel(
        _tab_kernel,
        mesh=mesh,
        compiler_params=pltpu.CompilerParams(use_tc_tiling_on_sc=False),
        out_type=[
            jax.ShapeDtypeStruct((B * NF, D), jnp.float32),
            jax.ShapeDtypeStruct((B, C), jnp.float32),
            jax.ShapeDtypeStruct((NC, NS, 2, L), jnp.float32),
        ],
        scratch_types=[
            pltpu.VMEM((NL,), jnp.int32),
            pltpu.VMEM((13 * L,), jnp.int32),
            pltpu.VMEM((NG, G), jnp.int32),
            pltpu.VMEM((ROWS_BUF, D), jnp.float32),
            pltpu.VMEM((NB * NC, C), jnp.float32),
            pltpu.VMEM((2, L), jnp.float32),
            pltpu.VMEM((NS, 2, L), jnp.float32),
            pltpu.VMEM((L,), jnp.float32),
            pltpu.VMEM((L,), jnp.float32),
            pltpu.SemaphoreType.DMA,
        ],
    )
    emb, bn, _ = f(xcat_flat, x_cont, table, gamma, beta)
    return jnp.concatenate([emb.reshape(B, NF * D), bn], axis=1)


def kernel(x_cat, x_cont, emb_tables, bn_gamma, bn_beta):
    xcat_flat = x_cat.astype(jnp.int32).reshape(B * NF)
    table = emb_tables.reshape(NF * V, D)
    return _run(xcat_flat, x_cont, table, bn_gamma, bn_beta)


# speedup vs baseline: 1.3006x; 1.1190x over previous
import jax
import jax.numpy as jnp
from jax import lax
from jax.experimental import pallas as pl
from jax.experimental.pallas import tpu as pltpu
from jax.experimental.pallas import tpu_sc as plsc

B = 16384
NF = 26
V = 100000
D = 32
C = 16

NC = 2
NS = 16
L = 16
NW = NC * NS
RPW = (NF * D) // NW
NB = B // NW
G = 128
NG = B // G


def _tab_kernel(tab_hbm, xT_hbm, xcont_hbm, gamma_hbm, beta_hbm,
                embT_hbm, bn_hbm, parts_hbm,
                xv, gidx, gbuf, xc, pbuf, pacc, gv, bv, sem):
    c = lax.axis_index("c")
    s = lax.axis_index("s")
    wid = s * NC + c

    pltpu.sync_copy(xcont_hbm.at[pl.ds(s * (NB * NC), NB * NC)], xc)

    def acc_body(i, carry):
        acc, acc2 = carry
        v = xc[i, :]
        return acc + v, acc2 + v * v

    zero = jnp.zeros((L,), jnp.float32)
    acc, acc2 = lax.fori_loop(0, NB * NC, acc_body, (zero, zero))
    pbuf[0, :] = acc
    pbuf[1, :] = acc2
    pltpu.sync_copy(pbuf, parts_hbm.at[c, s])
    plsc.subcore_barrier()

    pltpu.sync_copy(parts_hbm.at[c], pacc)
    tot = jnp.zeros((L,), jnp.float32)
    tot2 = jnp.zeros((L,), jnp.float32)
    for k in range(NS):
        tot = tot + pacc[k, 0, :]
        tot2 = tot2 + pacc[k, 1, :]
    inv_b = jnp.float32(1.0 / B)
    mean = tot * inv_b
    var = tot2 * inv_b - mean * mean
    x = var + jnp.float32(1e-5)
    i32 = lax.bitcast_convert_type(x, jnp.int32)
    i32 = jnp.int32(0x5F3759DF) - lax.shift_right_logical(i32, 1)
    y = lax.bitcast_convert_type(i32, jnp.float32)
    for _ in range(4):
        y = y * (jnp.float32(1.5) - jnp.float32(0.5) * x * y * y)
    pltpu.sync_copy(gamma_hbm, gv)
    pltpu.sync_copy(beta_hbm, bv)
    scale = gv[...] * y
    shift = bv[...] - mean * scale

    def bn_body(i, _):
        r = c * NB + i
        xc[r, :] = xc[r, :] * scale + shift
        return 0

    lax.fori_loop(0, NB, bn_body, 0)
    pltpu.sync_copy(xc.at[pl.ds(c * NB, NB)], bn_hbm.at[pl.ds(wid * NB, NB)])

    r0 = wid * RPW
    f1 = r0 // D
    n1 = jnp.minimum(RPW, (f1 + 1) * D - r0)

    def row_work(r, _):
        rv = r * V

        def idx_body(i, _):
            gidx[pl.ds(i * L, L)] = xv[pl.ds(i * L, L)] + rv
            return 0

        lax.fori_loop(0, B // L, idx_body, 0)

        def fire_body(g, _):
            pltpu.async_copy(tab_hbm.at[gidx.at[pl.ds(g * G, G)]],
                             gbuf.at[pl.ds(g * G, G)], sem)
            return 0

        lax.fori_loop(0, NG, fire_body, 0)
        pltpu.make_async_copy(tab_hbm.at[pl.ds(0, B)], gbuf, sem).wait()
        pltpu.sync_copy(gbuf, embT_hbm.at[r])
        return 0

    def seg(f, lo, n):
        @pl.when(n > 0)
        def _():
            pltpu.sync_copy(xT_hbm.at[f], xv)
            lax.fori_loop(lo, lo + n, row_work, 0)

    seg(f1, r0, n1)
    seg(f1 + 1, r0 + n1, RPW - n1)


@jax.jit
def _run(tabflat, xT, x_cont, gamma, beta):
    mesh = plsc.VectorSubcoreMesh(core_axis_name="c", subcore_axis_name="s")
    f = pl.kernel(
        _tab_kernel,
        mesh=mesh,
        compiler_params=pltpu.CompilerParams(use_tc_tiling_on_sc=False),
        out_type=[
            jax.ShapeDtypeStruct((NF * D, B), jnp.float32),
            jax.ShapeDtypeStruct((B, C), jnp.float32),
            jax.ShapeDtypeStruct((NC, NS, 2, L), jnp.float32),
        ],
        scratch_types=[
            pltpu.VMEM((B,), jnp.int32),
            pltpu.VMEM((B,), jnp.int32),
            pltpu.VMEM((B,), jnp.float32),
            pltpu.VMEM((NB * NC, C), jnp.float32),
            pltpu.VMEM((2, L), jnp.float32),
            pltpu.VMEM((NS, 2, L), jnp.float32),
            pltpu.VMEM((L,), jnp.float32),
            pltpu.VMEM((L,), jnp.float32),
            pltpu.SemaphoreType.DMA,
        ],
    )
    embT, bn, _ = f(tabflat, xT, x_cont, gamma, beta)
    return jnp.concatenate([embT.T, bn], axis=1)


def kernel(x_cat, x_cont, emb_tables, bn_gamma, bn_beta):
    tabflat = emb_tables.transpose(0, 2, 1).reshape(NF * D * V)
    xT = x_cat.astype(jnp.int32).T
    return _run(tabflat, xT, x_cont, bn_gamma, bn_beta)
